# Initial kernel scaffold; baseline (speedup 1.0000x reference)
import jax
import jax.numpy as jnp
from jax.experimental import pallas as pl


def kernel(x, edge_index, precomp, connection, radial1, phase1, b1, radial2, phase2, b2):
    raise NotImplementedError("write your pallas kernel here")


# speedup vs baseline: 22.3708x; 22.3708x over previous
import functools

import jax
import jax.numpy as jnp
from jax import lax
from jax.experimental import pallas as pl
from jax.experimental.pallas import tpu as pltpu
from jax.experimental.pallas import tpu_sc as plsc

N = 10000
E = 160000
C = 128
EPS = 1e-12

NSUB = 16
EB = 16
ECHUNK = E // NSUB
NBLK = ECHUNK // EB
NPAD = 10240
ROWS_PER_TILE = NPAD // NSUB
ZW = 2 * 2 * 2 * C

_f32 = jnp.float32


def _dotT(a, w):
    return lax.dot_general(a, w, dimension_numbers=(((1,), (1,)), ((), ())),
                           preferred_element_type=_f32,
                           precision=lax.Precision.HIGHEST)


def _wprep_body(rad_ref, ph_ref, wc_ref, ws_ref):
    rad = rad_ref[...]
    ph = ph_ref[...]
    wc_ref[...] = rad * jnp.cos(ph)[:, :, None]
    ws_ref[...] = rad * jnp.sin(ph)[:, :, None]


def _weight_prep(rads, phs):
    return pl.pallas_call(
        _wprep_body,
        out_shape=[jax.ShapeDtypeStruct((2, 4, 2, C, C), _f32)] * 2,
    )(rads, phs)


_EPB = 1000


def _eprep_body(pc_ref, cn_ref, fp_ref):
    pc = pc_ref[...]
    cn = cn_ref[...]
    cr, ci = cn[:, 0:1], cn[:, 1:2]

    def col(m_idx, r, p):
        j = m_idx * 4 + r * 2 + p
        return pc[:, j:j + 1]

    outs0, outs1 = [], []
    for r in (0, 1):
        a = col(1, r, 0)
        outs0 += [a, jnp.zeros_like(a)]
    for r in (0, 1):
        a, b = col(0, r, 0), -col(0, r, 1)
        outs0 += [a * cr - b * ci, a * ci + b * cr]
    for r in (0, 1):
        outs1 += [col(2, r, 0), col(2, r, 1)]
    for r in (0, 1):
        a = col(1, r, 0)
        outs1 += [a * cr, a * ci]
    pad = [jnp.zeros_like(outs0[0])] * 8
    fp_ref[0] = jnp.concatenate(outs0 + pad, axis=1)
    fp_ref[1] = jnp.concatenate(outs1 + pad, axis=1)


def _edge_prep(pc, cn):
    nb = E // _EPB
    return pl.pallas_call(
        _eprep_body,
        grid=(nb,),
        in_specs=[pl.BlockSpec((_EPB, 12), lambda i: (i, 0)),
                  pl.BlockSpec((_EPB, 2), lambda i: (i, 0))],
        out_specs=pl.BlockSpec((2, _EPB, 16), lambda i: (0, i, 0)),
        out_shape=jax.ShapeDtypeStruct((2, E, 16), _f32),
    )(pc, cn)


_NB = 1000


def _z_cols(xplanes, wc_ref, ws_ref, oo):
    cols = []
    for io in range(2):
        o = io * 2 + oo
        xr, xi = xplanes[io]
        for r in range(2):
            wc, ws = wc_ref[o, r], ws_ref[o, r]
            cols.append(_dotT(xr, wc) - _dotT(xi, ws))
            cols.append(_dotT(xr, ws) + _dotT(xi, wc))
    return jnp.concatenate(cols, axis=1)


def _z1_body(xt_ref, wc_ref, ws_ref, z_ref):
    xplanes = [(xt_ref[io, 0], xt_ref[io, 1]) for io in range(2)]
    for oo in range(2):
        z_ref[oo] = _z_cols(xplanes, wc_ref, ws_ref, oo)


def _z1(xt, wc, ws):
    return pl.pallas_call(
        _z1_body,
        grid=(N // _NB,),
        in_specs=[pl.BlockSpec((2, 2, _NB, C), lambda i: (0, 0, i, 0)),
                  pl.BlockSpec((4, 2, C, C), lambda i: (0, 0, 0, 0)),
                  pl.BlockSpec((4, 2, C, C), lambda i: (0, 0, 0, 0))],
        out_specs=pl.BlockSpec((2, _NB, ZW), lambda i: (0, i, 0)),
        out_shape=jax.ShapeDtypeStruct((2, N, ZW), _f32),
    )(xt, wc, ws)


def _nonlin(re, im, b):
    mag = jnp.sqrt(jnp.maximum(re * re + im * im, EPS))
    scale = jax.nn.relu(mag + b) / mag
    return scale * re, scale * im


def _z2_body(a_ref, b_ref, wc_ref, ws_ref, z_ref):
    b = b_ref[0]
    xplanes = []
    for m in range(2):
        xplanes.append(_nonlin(a_ref[0, m], a_ref[1, m], b[None, :]))
    for oo in range(2):
        z_ref[oo] = _z_cols(xplanes, wc_ref, ws_ref, oo)


def _z2(a, b, wc, ws):
    return pl.pallas_call(
        _z2_body,
        grid=(N // _NB,),
        in_specs=[pl.BlockSpec((2, 2, _NB, C), lambda i: (0, 0, i, 0)),
                  pl.BlockSpec((1, C), lambda i: (0, 0)),
                  pl.BlockSpec((4, 2, C, C), lambda i: (0, 0, 0, 0)),
                  pl.BlockSpec((4, 2, C, C), lambda i: (0, 0, 0, 0))],
        out_specs=pl.BlockSpec((2, _NB, ZW), lambda i: (0, i, 0)),
        out_shape=jax.ShapeDtypeStruct((2, N, ZW), _f32),
    )(a, b, wc, ws)


def _fin_body(a_ref, xt_ref, b_ref, y_ref):
    b = b_ref[0]
    for m in range(2):
        re = a_ref[0, m] + xt_ref[m, 0]
        im = a_ref[1, m] + xt_ref[m, 1]
        yr, yi = _nonlin(re, im, b[None, :])
        y_ref[0, m] = yr
        y_ref[1, m] = yi


def _fin(a, xt, b):
    return pl.pallas_call(
        _fin_body,
        grid=(N // _NB,),
        in_specs=[pl.BlockSpec((2, 2, _NB, C), lambda i: (0, 0, i, 0)),
                  pl.BlockSpec((2, 2, _NB, C), lambda i: (0, 0, i, 0)),
                  pl.BlockSpec((1, C), lambda i: (0, 0))],
        out_specs=pl.BlockSpec((2, 2, _NB, C), lambda i: (0, 0, i, 0)),
        out_shape=jax.ShapeDtypeStruct((2, 2, N, C), _f32),
    )(a, xt, b)


_sc_mesh = plsc.VectorSubcoreMesh(core_axis_name="c", subcore_axis_name="s")


@functools.partial(
    pl.kernel,
    out_type=jax.ShapeDtypeStruct((2, 2 * N, C), _f32),
    mesh=_sc_mesh,
    scratch_types=[
        pltpu.VMEM((EB, ZW), _f32),
        pltpu.VMEM((EB, C), _f32),
        pltpu.VMEM((EB,), jnp.int32),
        pltpu.VMEM((EB,), jnp.int32),
        pltpu.VMEM((EB, 16), _f32),
        pltpu.VMEM((8, C), _f32),
        pltpu.VMEM_SHARED((NPAD, C), _f32),
    ],
)
def _sc_conv(z_hbm, dst_hbm, src_hbm, fp_hbm, out_hbm,
             gbuf, msg, didx, sidx, fps, zbuf, acc):
    cid = lax.axis_index("c")
    sid = lax.axis_index("s")

    @pl.loop(0, 8)
    def _zr(r):
        @pl.loop(0, C, step=16)
        def _zc(j):
            zbuf[r, pl.ds(j, 16)] = jnp.zeros((16,), _f32)

    for p in (0, 1):
        @pl.loop(0, ROWS_PER_TILE // 8)
        def _zi(j):
            pltpu.sync_copy(zbuf, acc.at[pl.ds(sid * ROWS_PER_TILE + j * 8, 8)])

        plsc.subcore_barrier()

        @pl.loop(0, NBLK)
        def _blk(blk):
            e0 = sid * ECHUNK + blk * EB
            pltpu.sync_copy(dst_hbm.at[pl.ds(e0, EB)], didx)
            pltpu.sync_copy(src_hbm.at[pl.ds(e0, EB)], sidx)
            pltpu.sync_copy(fp_hbm.at[cid, pl.ds(e0, EB)], fps)

            @pl.loop(0, EB, step=16)
            def _off(j):
                didx[pl.ds(j, 16)] = didx[pl.ds(j, 16)] + cid * N

            pltpu.sync_copy(z_hbm.at[didx], gbuf)

            @pl.loop(0, EB)
            def _edge(i):
                srow = fps[i, pl.ds(0, 16)]
                s = [srow[t] for t in range(8)]
                for k in range(8):
                    m_acc = None
                    for combo in range(4):
                        io, r = combo >> 1, combo & 1
                        off = io * 512 + r * 256 + k * 16
                        gre = gbuf[i, pl.ds(off, 16)]
                        gim = gbuf[i, pl.ds(off + 128, 16)]
                        a, b = s[combo * 2], s[combo * 2 + 1]
                        if p == 0:
                            t_ = a * gre - b * gim
                        else:
                            t_ = a * gim + b * gre
                        m_acc = t_ if m_acc is None else m_acc + t_
                    msg[i, pl.ds(k * 16, 16)] = m_acc

            pltpu.sync_copy(msg, acc.at[sidx], add=True)

        plsc.subcore_barrier()
        @pl.when(sid < NSUB - 1)
        def _rb_full():
            pltpu.sync_copy(
                acc.at[pl.ds(sid * ROWS_PER_TILE, ROWS_PER_TILE)],
                out_hbm.at[p, pl.ds(cid * N + sid * ROWS_PER_TILE,
                                    ROWS_PER_TILE)])

        @pl.when(sid == NSUB - 1)
        def _rb_tail():
            pltpu.sync_copy(
                acc.at[pl.ds((NSUB - 1) * ROWS_PER_TILE,
                             N - (NSUB - 1) * ROWS_PER_TILE)],
                out_hbm.at[p, pl.ds(cid * N + (NSUB - 1) * ROWS_PER_TILE,
                                    N - (NSUB - 1) * ROWS_PER_TILE)])


def kernel(x, edge_index, precomp, connection,
           radial1, phase1, b1, radial2, phase2, b2):
    src = edge_index[0]
    dst = edge_index[1]
    xt = jnp.transpose(x, (1, 3, 0, 2))
    pc = precomp.reshape(E, 12)
    rads = jnp.stack([radial1, radial2])
    phs = jnp.stack([phase1, phase2])

    wc, ws = _weight_prep(rads, phs)
    fp = _edge_prep(pc, connection)

    z1 = _z1(xt, wc[0], ws[0]).reshape(2 * N, ZW)
    a1 = _sc_conv(z1, dst, src, fp)
    z2 = _z2(a1.reshape(2, 2, N, C), b1.reshape(1, C),
             wc[1], ws[1]).reshape(2 * N, ZW)
    a2 = _sc_conv(z2, dst, src, fp)
    yt = _fin(a2.reshape(2, 2, N, C), xt, b2.reshape(1, C))
    return jnp.transpose(yt, (2, 1, 3, 0))


# speedup vs baseline: 49.5884x; 2.2167x over previous
import functools

import jax
import jax.numpy as jnp
from jax import lax
from jax.experimental import pallas as pl
from jax.experimental.pallas import tpu as pltpu
from jax.experimental.pallas import tpu_sc as plsc

N = 10000
E = 160000
C = 128
EPS = 1e-12

NSUB = 16
EB = 16
ECHUNK = E // NSUB
SBE = 400
NSB = ECHUNK // SBE
NBLK_SB = SBE // EB
NPAD = 10240
ROWS_PER_TILE = NPAD // NSUB
ZW = 2 * 2 * 2 * C

_f32 = jnp.float32


def _dotT(a, w):
    return lax.dot_general(a, w, dimension_numbers=(((1,), (1,)), ((), ())),
                           preferred_element_type=_f32,
                           precision=lax.Precision.HIGHEST)


def _wprep_body(rad_ref, ph_ref, wc_ref, ws_ref):
    rad = rad_ref[...]
    ph = ph_ref[...]
    wc_ref[...] = rad * jnp.cos(ph)[:, :, None]
    ws_ref[...] = rad * jnp.sin(ph)[:, :, None]


def _weight_prep(rads, phs):
    return pl.pallas_call(
        _wprep_body,
        out_shape=[jax.ShapeDtypeStruct((2, 4, 2, C, C), _f32)] * 2,
    )(rads, phs)


_EPB = 1000


def _eprep_body(pc_ref, cn_ref, fp_ref):
    pc = pc_ref[...]
    cn = cn_ref[...]
    cr, ci = cn[:, 0:1], cn[:, 1:2]

    def col(m_idx, r, p):
        j = m_idx * 4 + r * 2 + p
        return pc[:, j:j + 1]

    outs0, outs1 = [], []
    for r in (0, 1):
        a = col(1, r, 0)
        outs0 += [a, jnp.zeros_like(a)]
    for r in (0, 1):
        a, b = col(0, r, 0), -col(0, r, 1)
        outs0 += [a * cr - b * ci, a * ci + b * cr]
    for r in (0, 1):
        outs1 += [col(2, r, 0), col(2, r, 1)]
    for r in (0, 1):
        a = col(1, r, 0)
        outs1 += [a * cr, a * ci]
    fp_ref[0] = jnp.concatenate(outs0, axis=1)
    fp_ref[1] = jnp.concatenate(outs1, axis=1)


def _edge_prep(pc, cn):
    nb = E // _EPB
    return pl.pallas_call(
        _eprep_body,
        grid=(nb,),
        in_specs=[pl.BlockSpec((_EPB, 12), lambda i: (i, 0)),
                  pl.BlockSpec((_EPB, 2), lambda i: (i, 0))],
        out_specs=pl.BlockSpec((2, _EPB, 8), lambda i: (0, i, 0)),
        out_shape=jax.ShapeDtypeStruct((2, E, 8), _f32),
    )(pc, cn)


_NB = 1000


def _z_cols(xplanes, wc_ref, ws_ref, oo):
    cols = []
    for io in range(2):
        o = io * 2 + oo
        xr, xi = xplanes[io]
        for r in range(2):
            wc, ws = wc_ref[o, r], ws_ref[o, r]
            cols.append(_dotT(xr, wc) - _dotT(xi, ws))
            cols.append(_dotT(xr, ws) + _dotT(xi, wc))
    return jnp.concatenate(cols, axis=1)


def _z1_body(xt_ref, wc_ref, ws_ref, z_ref):
    xplanes = [(xt_ref[io, 0], xt_ref[io, 1]) for io in range(2)]
    for oo in range(2):
        z_ref[oo] = _z_cols(xplanes, wc_ref, ws_ref, oo)


def _z1(xt, wc, ws):
    return pl.pallas_call(
        _z1_body,
        grid=(N // _NB,),
        in_specs=[pl.BlockSpec((2, 2, _NB, C), lambda i: (0, 0, i, 0)),
                  pl.BlockSpec((4, 2, C, C), lambda i: (0, 0, 0, 0)),
                  pl.BlockSpec((4, 2, C, C), lambda i: (0, 0, 0, 0))],
        out_specs=pl.BlockSpec((2, _NB, ZW), lambda i: (0, i, 0)),
        out_shape=jax.ShapeDtypeStruct((2, N, ZW), _f32),
    )(xt, wc, ws)


def _nonlin(re, im, b):
    mag = jnp.sqrt(jnp.maximum(re * re + im * im, EPS))
    scale = jax.nn.relu(mag + b) / mag
    return scale * re, scale * im


def _z2_body(a_ref, b_ref, wc_ref, ws_ref, z_ref):
    b = b_ref[0]
    xplanes = []
    for m in range(2):
        xplanes.append(_nonlin(a_ref[0, m], a_ref[1, m], b[None, :]))
    for oo in range(2):
        z_ref[oo] = _z_cols(xplanes, wc_ref, ws_ref, oo)


def _z2(a, b, wc, ws):
    return pl.pallas_call(
        _z2_body,
        grid=(N // _NB,),
        in_specs=[pl.BlockSpec((2, 2, _NB, C), lambda i: (0, 0, i, 0)),
                  pl.BlockSpec((1, C), lambda i: (0, 0)),
                  pl.BlockSpec((4, 2, C, C), lambda i: (0, 0, 0, 0)),
                  pl.BlockSpec((4, 2, C, C), lambda i: (0, 0, 0, 0))],
        out_specs=pl.BlockSpec((2, _NB, ZW), lambda i: (0, i, 0)),
        out_shape=jax.ShapeDtypeStruct((2, N, ZW), _f32),
    )(a, b, wc, ws)


def _fin_body(a_ref, xt_ref, b_ref, y_ref):
    b = b_ref[0]
    for m in range(2):
        re = a_ref[0, m] + xt_ref[m, 0]
        im = a_ref[1, m] + xt_ref[m, 1]
        yr, yi = _nonlin(re, im, b[None, :])
        y_ref[0, m] = yr
        y_ref[1, m] = yi


def _fin(a, xt, b):
    return pl.pallas_call(
        _fin_body,
        grid=(N // _NB,),
        in_specs=[pl.BlockSpec((2, 2, _NB, C), lambda i: (0, 0, i, 0)),
                  pl.BlockSpec((2, 2, _NB, C), lambda i: (0, 0, i, 0)),
                  pl.BlockSpec((1, C), lambda i: (0, 0))],
        out_specs=pl.BlockSpec((2, 2, _NB, C), lambda i: (0, 0, i, 0)),
        out_shape=jax.ShapeDtypeStruct((2, 2, N, C), _f32),
    )(a, xt, b)


_sc_mesh = plsc.VectorSubcoreMesh(core_axis_name="c", subcore_axis_name="s")


@functools.partial(
    pl.kernel,
    out_type=jax.ShapeDtypeStruct((2, 2 * N, C), _f32),
    mesh=_sc_mesh,
    scratch_types=[
        pltpu.VMEM((EB, ZW), _f32),
        pltpu.VMEM((EB, ZW), _f32),
        pltpu.VMEM((EB, C), _f32),
        pltpu.VMEM((EB, C), _f32),
        pltpu.VMEM((NBLK_SB, EB), jnp.int32),
        pltpu.VMEM((NBLK_SB, EB), jnp.int32),
        pltpu.VMEM((SBE * 8 + 16,), _f32),
        pltpu.VMEM_SHARED((NPAD, C), _f32),
        pltpu.SemaphoreType.DMA,
        pltpu.SemaphoreType.DMA,
        pltpu.SemaphoreType.DMA,
        pltpu.SemaphoreType.DMA,
    ],
)
def _sc_conv(z_hbm, dst_hbm, src_hbm, fp_hbm, out_hbm,
             gbufA, gbufB, msgA, msgB, didxS, sidxS, fpsF, acc,
             gsemA, gsemB, ssemA, ssemB):
    cid = lax.axis_index("c")
    sid = lax.axis_index("s")

    def wait_gather(gbuf, gsem):
        pltpu.make_async_copy(z_hbm.at[pl.ds(0, EB)], gbuf, gsem).wait()

    def wait_scatter(msg, ssem, p):
        pltpu.make_async_copy(out_hbm.at[p, pl.ds(0, EB)], msg, ssem).wait()

    def compute_block(j, gbuf, msg, p):
        @pl.loop(0, EB)
        def _edge(i):
            srow = fpsF[pl.ds((j * EB + i) * 8, 16)]
            s = [srow[t] for t in range(8)]
            for k in range(8):
                m_acc = None
                for combo in range(4):
                    io, r = combo >> 1, combo & 1
                    off = io * 512 + r * 256 + k * 16
                    gre = gbuf[i, pl.ds(off, 16)]
                    gim = gbuf[i, pl.ds(off + 128, 16)]
                    a, b = s[combo * 2], s[combo * 2 + 1]
                    if p == 0:
                        t_ = a * gre - b * gim
                    else:
                        t_ = a * gim + b * gre
                    m_acc = t_ if m_acc is None else m_acc + t_
                msg[i, pl.ds(k * 16, 16)] = m_acc

    for p in (0, 1):
        @pl.loop(0, EB)
        def _zr(r):
            @pl.loop(0, C, step=16)
            def _zc(j):
                msgA[r, pl.ds(j, 16)] = jnp.zeros((16,), _f32)

        @pl.loop(0, ROWS_PER_TILE // EB)
        def _zi(j):
            pltpu.sync_copy(msgA,
                            acc.at[pl.ds(sid * ROWS_PER_TILE + j * EB, EB)])

        plsc.subcore_barrier()

        @pl.loop(0, NSB)
        def _sb(s_):
            pltpu.sync_copy(dst_hbm.at[sid, s_], didxS)
            pltpu.sync_copy(src_hbm.at[sid, s_], sidxS)
            pltpu.sync_copy(
                fp_hbm.at[cid, pl.ds(sid * (ECHUNK * 8) + s_ * (SBE * 8),
                                     SBE * 8)],
                fpsF.at[pl.ds(0, SBE * 8)])

            @pl.loop(0, NBLK_SB)
            def _off(j):
                didxS[j, pl.ds(0, EB)] = didxS[j, pl.ds(0, EB)] + cid * N

            pltpu.async_copy(z_hbm.at[didxS.at[0]], gbufA, gsemA)
            pltpu.async_copy(z_hbm.at[didxS.at[1]], gbufB, gsemB)

            @pl.loop(0, (NBLK_SB - 1) // 2)
            def _pair(t):
                for par, (g, m, gs, ss) in enumerate(
                        ((gbufA, msgA, gsemA, ssemA),
                         (gbufB, msgB, gsemB, ssemB))):
                    j = t * 2 + par
                    wait_gather(g, gs)

                    @pl.when(t > 0)
                    def _ws():
                        wait_scatter(m, ss, p)

                    compute_block(j, g, m, p)
                    pltpu.async_copy(m, acc.at[sidxS.at[j]], ss, add=True)

                    @pl.when(j + 2 < NBLK_SB)
                    def _ig():
                        pltpu.async_copy(z_hbm.at[didxS.at[j + 2]], g, gs)

            wait_gather(gbufA, gsemA)
            wait_scatter(msgA, ssemA, p)
            compute_block(NBLK_SB - 1, gbufA, msgA, p)
            pltpu.async_copy(msgA, acc.at[sidxS.at[NBLK_SB - 1]], ssemA,
                             add=True)
            wait_scatter(msgA, ssemA, p)
            wait_scatter(msgB, ssemB, p)

        plsc.subcore_barrier()
        @pl.when(sid < NSUB - 1)
        def _rb_full():
            pltpu.sync_copy(
                acc.at[pl.ds(sid * ROWS_PER_TILE, ROWS_PER_TILE)],
                out_hbm.at[p, pl.ds(cid * N + sid * ROWS_PER_TILE,
                                    ROWS_PER_TILE)])

        @pl.when(sid == NSUB - 1)
        def _rb_tail():
            pltpu.sync_copy(
                acc.at[pl.ds((NSUB - 1) * ROWS_PER_TILE,
                             N - (NSUB - 1) * ROWS_PER_TILE)],
                out_hbm.at[p, pl.ds(cid * N + (NSUB - 1) * ROWS_PER_TILE,
                                    N - (NSUB - 1) * ROWS_PER_TILE)])


def kernel(x, edge_index, precomp, connection,
           radial1, phase1, b1, radial2, phase2, b2):
    src = edge_index[0]
    dst = edge_index[1]
    xt = jnp.transpose(x, (1, 3, 0, 2))
    pc = precomp.reshape(E, 12)
    rads = jnp.stack([radial1, radial2])
    phs = jnp.stack([phase1, phase2])

    wc, ws = _weight_prep(rads, phs)
    fp = _edge_prep(pc, connection)
    fpf = fp.reshape(2, E * 8)
    dst_r = dst.reshape(NSUB, NSB, NBLK_SB, EB)
    src_r = src.reshape(NSUB, NSB, NBLK_SB, EB)

    z1 = _z1(xt, wc[0], ws[0]).reshape(2 * N, ZW)
    a1 = _sc_conv(z1, dst_r, src_r, fpf)
    z2 = _z2(a1.reshape(2, 2, N, C), b1.reshape(1, C),
             wc[1], ws[1]).reshape(2 * N, ZW)
    a2 = _sc_conv(z2, dst_r, src_r, fpf)
    yt = _fin(a2.reshape(2, 2, N, C), xt, b2.reshape(1, C))
    return jnp.transpose(yt, (2, 1, 3, 0))


# speedup vs baseline: 49.6028x; 1.0003x over previous
import functools

import jax
import jax.numpy as jnp
from jax import lax
from jax.experimental import pallas as pl
from jax.experimental.pallas import tpu as pltpu
from jax.experimental.pallas import tpu_sc as plsc

N = 10000
E = 160000
C = 128
EPS = 1e-12

NSUB = 16
EB = 16
ECHUNK = E // NSUB
SBE = 400
NSB = ECHUNK // SBE
NBLK_SB = SBE // EB
NPAD = 10240
ROWS_PER_TILE = NPAD // NSUB
ZW = 2 * 2 * 2 * C

_f32 = jnp.float32


def _dotT(a, w):
    return lax.dot_general(a, w, dimension_numbers=(((1,), (1,)), ((), ())),
                           preferred_element_type=_f32,
                           precision=lax.Precision.HIGHEST)


def _wprep_body(rad_ref, ph_ref, wc_ref, ws_ref):
    rad = rad_ref[...]
    ph = ph_ref[...]
    wc_ref[...] = rad * jnp.cos(ph)[:, :, None]
    ws_ref[...] = rad * jnp.sin(ph)[:, :, None]


def _weight_prep(rads, phs):
    return pl.pallas_call(
        _wprep_body,
        out_shape=[jax.ShapeDtypeStruct((2, 4, 2, C, C), _f32)] * 2,
    )(rads, phs)


_EPB = 4000


def _eprep_body(pc_ref, cn_ref, fp_ref):
    pc = pc_ref[...]
    cn = cn_ref[...]
    cr, ci = cn[:, 0:1], cn[:, 1:2]

    def col(m_idx, r, p):
        j = m_idx * 4 + r * 2 + p
        return pc[:, j:j + 1]

    outs0, outs1 = [], []
    for r in (0, 1):
        a = col(1, r, 0)
        outs0 += [a, jnp.zeros_like(a)]
    for r in (0, 1):
        a, b = col(0, r, 0), -col(0, r, 1)
        outs0 += [a * cr - b * ci, a * ci + b * cr]
    for r in (0, 1):
        outs1 += [col(2, r, 0), col(2, r, 1)]
    for r in (0, 1):
        a = col(1, r, 0)
        outs1 += [a * cr, a * ci]
    fp_ref[0] = jnp.concatenate(outs0, axis=1)
    fp_ref[1] = jnp.concatenate(outs1, axis=1)


def _edge_prep(pc, cn):
    nb = E // _EPB
    return pl.pallas_call(
        _eprep_body,
        grid=(nb,),
        in_specs=[pl.BlockSpec((_EPB, 12), lambda i: (i, 0)),
                  pl.BlockSpec((_EPB, 2), lambda i: (i, 0))],
        out_specs=pl.BlockSpec((2, _EPB, 8), lambda i: (0, i, 0)),
        out_shape=jax.ShapeDtypeStruct((2, E, 8), _f32),
    )(pc, cn)


_NB = 1000


def _z_cols(xplanes, wc_ref, ws_ref, oo):
    cols = []
    for io in range(2):
        o = io * 2 + oo
        xr, xi = xplanes[io]
        for r in range(2):
            wc, ws = wc_ref[o, r], ws_ref[o, r]
            cols.append(_dotT(xr, wc) - _dotT(xi, ws))
            cols.append(_dotT(xr, ws) + _dotT(xi, wc))
    return jnp.concatenate(cols, axis=1)


def _z1_body(xt_ref, wc_ref, ws_ref, z_ref):
    xplanes = [(xt_ref[io, 0], xt_ref[io, 1]) for io in range(2)]
    for oo in range(2):
        z_ref[oo] = _z_cols(xplanes, wc_ref, ws_ref, oo)


def _z1(xt, wc, ws):
    return pl.pallas_call(
        _z1_body,
        grid=(N // _NB,),
        in_specs=[pl.BlockSpec((2, 2, _NB, C), lambda i: (0, 0, i, 0)),
                  pl.BlockSpec((4, 2, C, C), lambda i: (0, 0, 0, 0)),
                  pl.BlockSpec((4, 2, C, C), lambda i: (0, 0, 0, 0))],
        out_specs=pl.BlockSpec((2, _NB, ZW), lambda i: (0, i, 0)),
        out_shape=jax.ShapeDtypeStruct((2, N, ZW), _f32),
    )(xt, wc, ws)


def _nonlin(re, im, b):
    mag = jnp.sqrt(jnp.maximum(re * re + im * im, EPS))
    scale = jax.nn.relu(mag + b) / mag
    return scale * re, scale * im


def _z2_body(a_ref, b_ref, wc_ref, ws_ref, z_ref):
    b = b_ref[0]
    xplanes = []
    for m in range(2):
        xplanes.append(_nonlin(a_ref[0, m], a_ref[1, m], b[None, :]))
    for oo in range(2):
        z_ref[oo] = _z_cols(xplanes, wc_ref, ws_ref, oo)


def _z2(a, b, wc, ws):
    return pl.pallas_call(
        _z2_body,
        grid=(N // _NB,),
        in_specs=[pl.BlockSpec((2, 2, _NB, C), lambda i: (0, 0, i, 0)),
                  pl.BlockSpec((1, C), lambda i: (0, 0)),
                  pl.BlockSpec((4, 2, C, C), lambda i: (0, 0, 0, 0)),
                  pl.BlockSpec((4, 2, C, C), lambda i: (0, 0, 0, 0))],
        out_specs=pl.BlockSpec((2, _NB, ZW), lambda i: (0, i, 0)),
        out_shape=jax.ShapeDtypeStruct((2, N, ZW), _f32),
    )(a, b, wc, ws)


def _fin_body(a_ref, xt_ref, b_ref, y_ref):
    b = b_ref[0]
    for m in range(2):
        re = a_ref[0, m] + xt_ref[m, 0]
        im = a_ref[1, m] + xt_ref[m, 1]
        yr, yi = _nonlin(re, im, b[None, :])
        y_ref[0, m] = yr
        y_ref[1, m] = yi


def _fin(a, xt, b):
    return pl.pallas_call(
        _fin_body,
        grid=(N // _NB,),
        in_specs=[pl.BlockSpec((2, 2, _NB, C), lambda i: (0, 0, i, 0)),
                  pl.BlockSpec((2, 2, _NB, C), lambda i: (0, 0, i, 0)),
                  pl.BlockSpec((1, C), lambda i: (0, 0))],
        out_specs=pl.BlockSpec((2, 2, _NB, C), lambda i: (0, 0, i, 0)),
        out_shape=jax.ShapeDtypeStruct((2, 2, N, C), _f32),
    )(a, xt, b)


_sc_mesh = plsc.VectorSubcoreMesh(core_axis_name="c", subcore_axis_name="s")


@functools.partial(
    pl.kernel,
    out_type=jax.ShapeDtypeStruct((2, 2 * N, C), _f32),
    mesh=_sc_mesh,
    scratch_types=[
        pltpu.VMEM((EB, ZW), _f32),
        pltpu.VMEM((EB, ZW), _f32),
        pltpu.VMEM((EB, C), _f32),
        pltpu.VMEM((EB, C), _f32),
        pltpu.VMEM((NBLK_SB, EB), jnp.int32),
        pltpu.VMEM((NBLK_SB, EB), jnp.int32),
        pltpu.VMEM((SBE * 8 + 16,), _f32),
        pltpu.VMEM_SHARED((NPAD, C), _f32),
        pltpu.SemaphoreType.DMA,
        pltpu.SemaphoreType.DMA,
        pltpu.SemaphoreType.DMA,
        pltpu.SemaphoreType.DMA,
    ],
)
def _sc_conv(z_hbm, dst_hbm, src_hbm, fp_hbm, out_hbm,
             gbufA, gbufB, msgA, msgB, didxS, sidxS, fpsF, acc,
             gsemA, gsemB, ssemA, ssemB):
    cid = lax.axis_index("c")
    sid = lax.axis_index("s")

    def wait_gather(gbuf, gsem):
        pltpu.make_async_copy(z_hbm.at[pl.ds(0, EB)], gbuf, gsem).wait()

    def wait_scatter(msg, ssem, p):
        pltpu.make_async_copy(out_hbm.at[p, pl.ds(0, EB)], msg, ssem).wait()

    def compute_block(j, gbuf, msg, p):
        @pl.loop(0, EB)
        def _edge(i):
            srow = fpsF[pl.ds((j * EB + i) * 8, 16)]
            s = [srow[t] for t in range(8)]
            for k in range(8):
                m_acc = None
                for combo in range(4):
                    io, r = combo >> 1, combo & 1
                    off = io * 512 + r * 256 + k * 16
                    gre = gbuf[i, pl.ds(off, 16)]
                    gim = gbuf[i, pl.ds(off + 128, 16)]
                    a, b = s[combo * 2], s[combo * 2 + 1]
                    if p == 0:
                        t_ = a * gre - b * gim
                    else:
                        t_ = a * gim + b * gre
                    m_acc = t_ if m_acc is None else m_acc + t_
                msg[i, pl.ds(k * 16, 16)] = m_acc

    for p in (0, 1):
        @pl.loop(0, EB)
        def _zr(r):
            @pl.loop(0, C, step=16)
            def _zc(j):
                msgA[r, pl.ds(j, 16)] = jnp.zeros((16,), _f32)

        @pl.loop(0, ROWS_PER_TILE // EB)
        def _zi(j):
            pltpu.sync_copy(msgA,
                            acc.at[pl.ds(sid * ROWS_PER_TILE + j * EB, EB)])

        plsc.subcore_barrier()

        @pl.loop(0, NSB)
        def _sb(s_):
            pltpu.sync_copy(dst_hbm.at[sid, s_], didxS)
            pltpu.sync_copy(src_hbm.at[sid, s_], sidxS)
            pltpu.sync_copy(
                fp_hbm.at[cid, pl.ds(sid * (ECHUNK * 8) + s_ * (SBE * 8),
                                     SBE * 8)],
                fpsF.at[pl.ds(0, SBE * 8)])

            @pl.loop(0, NBLK_SB)
            def _off(j):
                didxS[j, pl.ds(0, EB)] = didxS[j, pl.ds(0, EB)] + cid * N

            pltpu.async_copy(z_hbm.at[didxS.at[0]], gbufA, gsemA)
            pltpu.async_copy(z_hbm.at[didxS.at[1]], gbufB, gsemB)

            @pl.loop(0, (NBLK_SB - 1) // 2)
            def _pair(t):
                for par, (g, m, gs, ss) in enumerate(
                        ((gbufA, msgA, gsemA, ssemA),
                         (gbufB, msgB, gsemB, ssemB))):
                    j = t * 2 + par
                    wait_gather(g, gs)

                    @pl.when(t > 0)
                    def _ws():
                        wait_scatter(m, ss, p)

                    compute_block(j, g, m, p)
                    pltpu.async_copy(m, acc.at[sidxS.at[j]], ss, add=True)

                    @pl.when(j + 2 < NBLK_SB)
                    def _ig():
                        pltpu.async_copy(z_hbm.at[didxS.at[j + 2]], g, gs)

            wait_gather(gbufA, gsemA)
            wait_scatter(msgA, ssemA, p)
            compute_block(NBLK_SB - 1, gbufA, msgA, p)
            pltpu.async_copy(msgA, acc.at[sidxS.at[NBLK_SB - 1]], ssemA,
                             add=True)
            wait_scatter(msgA, ssemA, p)
            wait_scatter(msgB, ssemB, p)

        plsc.subcore_barrier()
        @pl.when(sid < NSUB - 1)
        def _rb_full():
            pltpu.sync_copy(
                acc.at[pl.ds(sid * ROWS_PER_TILE, ROWS_PER_TILE)],
                out_hbm.at[p, pl.ds(cid * N + sid * ROWS_PER_TILE,
                                    ROWS_PER_TILE)])

        @pl.when(sid == NSUB - 1)
        def _rb_tail():
            pltpu.sync_copy(
                acc.at[pl.ds((NSUB - 1) * ROWS_PER_TILE,
                             N - (NSUB - 1) * ROWS_PER_TILE)],
                out_hbm.at[p, pl.ds(cid * N + (NSUB - 1) * ROWS_PER_TILE,
                                    N - (NSUB - 1) * ROWS_PER_TILE)])


def kernel(x, edge_index, precomp, connection,
           radial1, phase1, b1, radial2, phase2, b2):
    src = edge_index[0]
    dst = edge_index[1]
    xt = jnp.transpose(x, (1, 3, 0, 2))
    pc = precomp.reshape(E, 12)
    rads = jnp.stack([radial1, radial2])
    phs = jnp.stack([phase1, phase2])

    wc, ws = _weight_prep(rads, phs)
    fp = _edge_prep(pc, connection)
    fpf = fp.reshape(2, E * 8)
    dst_r = dst.reshape(NSUB, NSB, NBLK_SB, EB)
    src_r = src.reshape(NSUB, NSB, NBLK_SB, EB)

    z1 = _z1(xt, wc[0], ws[0]).reshape(2 * N, ZW)
    a1 = _sc_conv(z1, dst_r, src_r, fpf)
    z2 = _z2(a1.reshape(2, 2, N, C), b1.reshape(1, C),
             wc[1], ws[1]).reshape(2 * N, ZW)
    a2 = _sc_conv(z2, dst_r, src_r, fpf)
    yt = _fin(a2.reshape(2, 2, N, C), xt, b2.reshape(1, C))
    return jnp.transpose(yt, (2, 1, 3, 0))
